# speedup vs baseline: 1.2982x; 1.2982x over previous
import functools

import jax
import jax.numpy as jnp
from jax import lax
from jax.experimental import pallas as pl
from jax.experimental.pallas import tpu as pltpu
from jax.experimental.pallas import tpu_sc as plsc

NUM_EMBEDDINGS = 100000
EMBEDDING_DIM = 128
NUM_FEATURES = 26
BATCH_SIZE = 4096
TOTAL_ROWS = NUM_FEATURES * BATCH_SIZE

_INFO = plsc.get_sparse_core_info()
_NC = _INFO.num_cores
_NS = _INFO.num_subcores
_NW = _NC * _NS
ROWS_PER_WORKER = TOTAL_ROWS // _NW
CHUNK = 128
NCHUNKS = ROWS_PER_WORKER // CHUNK


def _gather_body(idx_hbm, table_hbm, out_hbm, idx_v, rows_v, sem):
    wid = lax.axis_index("s") * _NC + lax.axis_index("c")
    base = wid * ROWS_PER_WORKER
    pltpu.sync_copy(idx_hbm.at[wid], idx_v)

    def body(c, carry):
        pltpu.async_copy(table_hbm.at[idx_v.at[c]], rows_v, sem).wait()
        pltpu.sync_copy(rows_v, out_hbm.at[pl.ds(base + c * CHUNK, CHUNK)])
        return carry

    lax.fori_loop(0, NCHUNKS, body, 0)


@jax.jit
def _gather(idx, table):
    mesh = plsc.VectorSubcoreMesh(core_axis_name="c", subcore_axis_name="s")
    k = functools.partial(
        pl.kernel,
        mesh=mesh,
        out_type=jax.ShapeDtypeStruct((TOTAL_ROWS, EMBEDDING_DIM), jnp.float32),
        scratch_types=[
            pltpu.VMEM((NCHUNKS, CHUNK), jnp.int32),
            pltpu.VMEM((CHUNK, EMBEDDING_DIM), jnp.float32),
            pltpu.SemaphoreType.DMA,
        ],
    )(_gather_body)
    return k(idx, table)


def kernel(indices, lengths, weights):
    del lengths
    idx = indices.astype(jnp.int32).reshape(_NW, NCHUNKS, CHUNK)
    out = _gather(idx, weights)
    return out.reshape(NUM_FEATURES, BATCH_SIZE, EMBEDDING_DIM)


# speedup vs baseline: 1.4737x; 1.1351x over previous
import functools

import jax
import jax.numpy as jnp
from jax import lax
from jax.experimental import pallas as pl
from jax.experimental.pallas import tpu as pltpu
from jax.experimental.pallas import tpu_sc as plsc

NUM_EMBEDDINGS = 100000
EMBEDDING_DIM = 128
NUM_FEATURES = 26
BATCH_SIZE = 4096
TOTAL_ROWS = NUM_FEATURES * BATCH_SIZE

_INFO = plsc.get_sparse_core_info()
_NC = _INFO.num_cores
_NS = _INFO.num_subcores
_NW = _NC * _NS
ROWS_PER_WORKER = TOTAL_ROWS // _NW
CHUNK = 128
NCHUNKS = ROWS_PER_WORKER // CHUNK


def _gather_body(idx_hbm, table_hbm, out_hbm, idx_v, rows0, rows1, sem0, sem1):
    wid = lax.axis_index("s") * _NC + lax.axis_index("c")
    base = wid * ROWS_PER_WORKER
    pltpu.sync_copy(idx_hbm.at[wid], idx_v)
    bufs = (rows0, rows1)
    sems = (sem0, sem1)

    pltpu.async_copy(table_hbm.at[idx_v.at[0]], bufs[0], sems[0])

    def body(g, carry):
        for b in range(2):
            c = 2 * g + b
            nb = 1 - b
            pltpu.make_async_copy(table_hbm.at[idx_v.at[c]], bufs[b], sems[b]).wait()

            @pl.when(c + 1 < NCHUNKS)
            def _():
                pltpu.async_copy(table_hbm.at[idx_v.at[c + 1]], bufs[nb], sems[nb])

            pltpu.sync_copy(bufs[b], out_hbm.at[pl.ds(base + c * CHUNK, CHUNK)])
        return carry

    lax.fori_loop(0, NCHUNKS // 2, body, 0)


@jax.jit
def _gather(idx, table):
    mesh = plsc.VectorSubcoreMesh(core_axis_name="c", subcore_axis_name="s")
    k = functools.partial(
        pl.kernel,
        mesh=mesh,
        out_type=jax.ShapeDtypeStruct((TOTAL_ROWS, EMBEDDING_DIM), jnp.float32),
        scratch_types=[
            pltpu.VMEM((NCHUNKS, CHUNK), jnp.int32),
            pltpu.VMEM((CHUNK, EMBEDDING_DIM), jnp.float32),
            pltpu.VMEM((CHUNK, EMBEDDING_DIM), jnp.float32),
            pltpu.SemaphoreType.DMA,
            pltpu.SemaphoreType.DMA,
        ],
    )(_gather_body)
    return k(idx, table)


def kernel(indices, lengths, weights):
    del lengths
    idx = indices.astype(jnp.int32).reshape(_NW, NCHUNKS, CHUNK)
    out = _gather(idx, weights)
    return out.reshape(NUM_FEATURES, BATCH_SIZE, EMBEDDING_DIM)


# speedup vs baseline: 1.7320x; 1.1753x over previous
import functools

import jax
import jax.numpy as jnp
from jax import lax
from jax.experimental import pallas as pl
from jax.experimental.pallas import tpu as pltpu
from jax.experimental.pallas import tpu_sc as plsc

NUM_EMBEDDINGS = 100000
EMBEDDING_DIM = 128
NUM_FEATURES = 26
BATCH_SIZE = 4096
TOTAL_ROWS = NUM_FEATURES * BATCH_SIZE

_INFO = plsc.get_sparse_core_info()
_NC = _INFO.num_cores
_NS = _INFO.num_subcores
_NW = _NC * _NS
ROWS_PER_WORKER = TOTAL_ROWS // _NW
CHUNK = 104
NCHUNKS = ROWS_PER_WORKER // CHUNK
NBUF = 4


def _gather_body(idx_hbm, table_hbm, out_hbm, idx_v,
                 rows0, rows1, rows2, rows3,
                 g0, g1, g2, g3, w0, w1, w2, w3):
    wid = lax.axis_index("s") * _NC + lax.axis_index("c")
    base = wid * ROWS_PER_WORKER
    bufs = (rows0, rows1, rows2, rows3)
    gsem = (g0, g1, g2, g3)
    wsem = (w0, w1, w2, w3)

    pltpu.sync_copy(idx_hbm.at[pl.ds(base, ROWS_PER_WORKER)], idx_v)

    def gather(c, b):
        return pltpu.async_copy(
            table_hbm.at[idx_v.at[pl.ds(c * CHUNK, CHUNK)]], bufs[b], gsem[b])

    def gather_wait(c, b):
        pltpu.make_async_copy(
            table_hbm.at[idx_v.at[pl.ds(c * CHUNK, CHUNK)]], bufs[b], gsem[b]
        ).wait()

    def write(c, b):
        return pltpu.async_copy(
            bufs[b], out_hbm.at[pl.ds(base + c * CHUNK, CHUNK)], wsem[b])

    def write_wait(c, b):
        pltpu.make_async_copy(
            bufs[b], out_hbm.at[pl.ds(base + c * CHUNK, CHUNK)], wsem[b]
        ).wait()

    for b in range(NBUF - 1):
        gather(b, b)

    def body(g, carry):
        for b in range(NBUF):
            c = NBUF * g + b
            nb = (b + NBUF - 1) % NBUF
            nxt = c + NBUF - 1
            gather_wait(c, b)
            write(c, b)
            if b == 0:
                @pl.when(g > 0)
                def _():
                    write_wait(c - 1, nb)

                gather(nxt, nb)
            else:
                @pl.when(nxt < NCHUNKS)
                def _():
                    write_wait(c - 1, nb)
                    gather(nxt, nb)

        return carry

    lax.fori_loop(0, NCHUNKS // NBUF, body, 0)

    for b in range(NBUF):
        write_wait(NCHUNKS - NBUF + b, b)


@jax.jit
def _gather(idx, table):
    mesh = plsc.VectorSubcoreMesh(core_axis_name="c", subcore_axis_name="s")
    k = functools.partial(
        pl.kernel,
        mesh=mesh,
        out_type=jax.ShapeDtypeStruct((TOTAL_ROWS, EMBEDDING_DIM), jnp.float32),
        scratch_types=[
            pltpu.VMEM((ROWS_PER_WORKER,), jnp.int32),
        ]
        + [pltpu.VMEM((CHUNK, EMBEDDING_DIM), jnp.float32)] * NBUF
        + [pltpu.SemaphoreType.DMA] * (2 * NBUF),
    )(_gather_body)
    return k(idx, table)


def kernel(indices, lengths, weights):
    del lengths
    idx = indices.astype(jnp.int32).reshape(-1)
    out = _gather(idx, weights)
    return out.reshape(NUM_FEATURES, BATCH_SIZE, EMBEDDING_DIM)


# speedup vs baseline: 1.7360x; 1.0023x over previous
import functools

import jax
import jax.numpy as jnp
from jax import lax
from jax.experimental import pallas as pl
from jax.experimental.pallas import tpu as pltpu
from jax.experimental.pallas import tpu_sc as plsc

NUM_EMBEDDINGS = 100000
EMBEDDING_DIM = 128
NUM_FEATURES = 26
BATCH_SIZE = 4096
TOTAL_ROWS = NUM_FEATURES * BATCH_SIZE

_INFO = plsc.get_sparse_core_info()
_NC = _INFO.num_cores
_NS = _INFO.num_subcores
_NW = _NC * _NS
CHUNK = BATCH_SIZE // _NW
NCHUNKS = NUM_FEATURES
NBUF = 4
_MAIN = (NCHUNKS // NBUF) * NBUF


def _gather_body(idx_hbm, table_hbm, out_hbm, idx_v,
                 rows0, rows1, rows2, rows3,
                 g0, g1, g2, g3, w0, w1, w2, w3):
    wid = lax.axis_index("s") * _NC + lax.axis_index("c")
    col = wid * CHUNK
    bufs = (rows0, rows1, rows2, rows3)
    gsem = (g0, g1, g2, g3)
    wsem = (w0, w1, w2, w3)

    pltpu.sync_copy(idx_hbm.at[:, pl.ds(col, CHUNK)], idx_v)

    def gather(f, b):
        return pltpu.async_copy(table_hbm.at[idx_v.at[f]], bufs[b], gsem[b])

    def gather_wait(f, b):
        pltpu.make_async_copy(
            table_hbm.at[idx_v.at[f]], bufs[b], gsem[b]).wait()

    def write(f, b):
        return pltpu.async_copy(
            bufs[b], out_hbm.at[pl.ds(f * BATCH_SIZE + col, CHUNK)], wsem[b])

    def write_wait(f, b):
        pltpu.make_async_copy(
            bufs[b], out_hbm.at[pl.ds(f * BATCH_SIZE + col, CHUNK)], wsem[b]
        ).wait()

    for b in range(NBUF - 1):
        gather(b, b)

    def body(g, carry):
        for b in range(NBUF):
            f = NBUF * g + b
            nb = (b + NBUF - 1) % NBUF
            nxt = f + NBUF - 1
            gather_wait(f, b)
            write(f, b)
            if b == 0:
                @pl.when(g > 0)
                def _():
                    write_wait(f - 1, nb)

                gather(nxt, nb)
            elif b == NBUF - 1:
                @pl.when(nxt < NCHUNKS)
                def _():
                    write_wait(f - 1, nb)
                    gather(nxt, nb)
            else:
                write_wait(f - 1, nb)
                gather(nxt, nb)

        return carry

    lax.fori_loop(0, _MAIN // NBUF, body, 0)

    for f in range(_MAIN, NCHUNKS):
        b = f % NBUF
        gather_wait(f, b)
        write(f, b)

    for f in range(NCHUNKS - NBUF, NCHUNKS):
        write_wait(f, f % NBUF)


@jax.jit
def _gather(idx, table):
    mesh = plsc.VectorSubcoreMesh(core_axis_name="c", subcore_axis_name="s")
    k = functools.partial(
        pl.kernel,
        mesh=mesh,
        out_type=jax.ShapeDtypeStruct((TOTAL_ROWS, EMBEDDING_DIM), jnp.float32),
        scratch_types=[
            pltpu.VMEM((NCHUNKS, CHUNK), jnp.int32),
        ]
        + [pltpu.VMEM((CHUNK, EMBEDDING_DIM), jnp.float32)] * NBUF
        + [pltpu.SemaphoreType.DMA] * (2 * NBUF),
    )(_gather_body)
    return k(idx, table)


def kernel(indices, lengths, weights):
    del lengths
    out = _gather(indices.astype(jnp.int32), weights)
    return out.reshape(NUM_FEATURES, BATCH_SIZE, EMBEDDING_DIM)


# speedup vs baseline: 1.7669x; 1.0178x over previous
import functools

import jax
import jax.numpy as jnp
from jax import lax
from jax.experimental import pallas as pl
from jax.experimental.pallas import tpu as pltpu
from jax.experimental.pallas import tpu_sc as plsc

NUM_EMBEDDINGS = 100000
EMBEDDING_DIM = 128
NUM_FEATURES = 26
BATCH_SIZE = 4096
TOTAL_ROWS = NUM_FEATURES * BATCH_SIZE

_INFO = plsc.get_sparse_core_info()
_NC = _INFO.num_cores
_NS = _INFO.num_subcores
_NW = _NC * _NS
CHUNK = BATCH_SIZE // _NW
NCHUNKS = NUM_FEATURES
NBUF = 6
_MAIN = (NCHUNKS // NBUF) * NBUF


def _gather_body(idx_hbm, table_hbm, out_hbm, idx_v, *scratch):
    bufs = scratch[:NBUF]
    gsem = scratch[NBUF:2 * NBUF]
    wsem = scratch[2 * NBUF:]
    wid = lax.axis_index("s") * _NC + lax.axis_index("c")
    col = wid * CHUNK

    pltpu.sync_copy(idx_hbm.at[:, pl.ds(col, CHUNK)], idx_v)

    def gather(f, b):
        return pltpu.async_copy(table_hbm.at[idx_v.at[f]], bufs[b], gsem[b])

    def gather_wait(f, b):
        pltpu.make_async_copy(
            table_hbm.at[idx_v.at[f]], bufs[b], gsem[b]).wait()

    def write(f, b):
        return pltpu.async_copy(
            bufs[b], out_hbm.at[pl.ds(f * BATCH_SIZE + col, CHUNK)], wsem[b])

    def write_wait(f, b):
        pltpu.make_async_copy(
            bufs[b], out_hbm.at[pl.ds(f * BATCH_SIZE + col, CHUNK)], wsem[b]
        ).wait()

    for b in range(NBUF - 1):
        gather(b, b)

    def body(g, carry):
        for b in range(NBUF):
            f = NBUF * g + b
            nb = (b + NBUF - 1) % NBUF
            nxt = f + NBUF - 1
            gather_wait(f, b)
            write(f, b)
            if b == 0:
                @pl.when(g > 0)
                def _():
                    write_wait(f - 1, nb)

                gather(nxt, nb)
            else:
                @pl.when(nxt < NCHUNKS)
                def _():
                    write_wait(f - 1, nb)
                    gather(nxt, nb)

        return carry

    lax.fori_loop(0, _MAIN // NBUF, body, 0)

    for f in range(_MAIN, NCHUNKS):
        gather_wait(f, f % NBUF)
        write(f, f % NBUF)

    for f in range(NCHUNKS - NBUF, NCHUNKS):
        write_wait(f, f % NBUF)


@jax.jit
def _gather(idx, table):
    mesh = plsc.VectorSubcoreMesh(core_axis_name="c", subcore_axis_name="s")
    k = functools.partial(
        pl.kernel,
        mesh=mesh,
        out_type=jax.ShapeDtypeStruct((TOTAL_ROWS, EMBEDDING_DIM), jnp.float32),
        scratch_types=[
            pltpu.VMEM((NCHUNKS, CHUNK), jnp.int32),
        ]
        + [pltpu.VMEM((CHUNK, EMBEDDING_DIM), jnp.float32)] * NBUF
        + [pltpu.SemaphoreType.DMA] * (2 * NBUF),
    )(_gather_body)
    return k(idx, table)


def kernel(indices, lengths, weights):
    del lengths
    out = _gather(indices.astype(jnp.int32), weights)
    return out.reshape(NUM_FEATURES, BATCH_SIZE, EMBEDDING_DIM)
